# baseline (speedup 1.0000x reference)
import functools

import jax
import jax.numpy as jnp
from jax import lax
from jax.experimental import pallas as pl
from jax.experimental.pallas import tpu as pltpu
from jax.experimental.pallas import tpu_sc as plsc

N = 10000
NP = 10240
E = 320000
ER = E // 128
C = 128
RB = 1024


def _mesh():
    return plsc.VectorSubcoreMesh(core_axis_name="c", subcore_axis_name="s")


ROWS_T = 78
CH = 3
NCH = ROWS_T // CH
NPAIR = NCH // 2 - 1


def _deg_kernel(e4):
    @functools.partial(
        pl.kernel,
        mesh=_mesh(),
        out_type=jax.ShapeDtypeStruct((2, NP, C), jnp.float32),
        scratch_types=[
            pltpu.VMEM_SHARED((NP, C), jnp.float32),
            pltpu.VMEM((128, C), jnp.float32),
            pltpu.VMEM((80, 1, 128), jnp.int32),
            pltpu.SemaphoreType.DMA,
            pltpu.SemaphoreType.DMA,
        ],
    )
    def k(ei_hbm, zdum_hbm, out_hbm, acc, buf, didx, sem0, sem1):
        c = lax.axis_index("c")
        s = lax.axis_index("s")
        sems = (sem0, sem1)
        zeroi = jnp.zeros((16,), jnp.float32)
        onesi = jnp.ones((16,), jnp.float32)

        def zb(i, _):
            buf[i // 8, pl.ds((i % 8) * 16, 16)] = zeroi
            return 0

        lax.fori_loop(0, 1024, zb, 0)
        for j in range(5):
            pltpu.sync_copy(buf, acc.at[pl.ds(s * 640 + j * 128, 128)])

        def ob(i, _):
            buf[i // 8, pl.ds((i % 8) * 16, 16)] = onesi
            return 0

        lax.fori_loop(0, 1024, ob, 0)
        base = c * (ER // 2) + s * ROWS_T
        pltpu.sync_copy(ei_hbm.at[1, pl.ds(base, ROWS_T)],
                        didx.at[pl.ds(0, ROWS_T)])

        @pl.when(s < 2)
        def _():
            pltpu.sync_copy(ei_hbm.at[1, c * (ER // 2) + 16 * ROWS_T + s],
                            didx.at[ROWS_T])

        plsc.subcore_barrier()

        def scat(j, b):
            for kk in range(CH):
                pltpu.async_copy(buf, acc.at[didx.at[j * CH + kk, 0]],
                                 sems[b], add=True)

        def wait_s(b):
            for kk in range(CH):
                pltpu.make_async_copy(zdum_hbm.at[pl.ds(0, 128)], buf,
                                      sems[b]).wait()

        scat(0, 0)
        scat(1, 1)

        def pair(j2, _):
            j = 2 * j2
            wait_s(0)
            scat(j + 2, 0)
            wait_s(1)
            scat(j + 3, 1)
            return 0

        lax.fori_loop(0, NPAIR, pair, 0)
        wait_s(0)
        wait_s(1)

        @pl.when(s < 2)
        def _():
            pltpu.async_copy(buf, acc.at[didx.at[ROWS_T, 0]], sem0, add=True)
            pltpu.make_async_copy(zdum_hbm.at[pl.ds(0, 128)], buf,
                                  sem0).wait()

        plsc.subcore_barrier()
        pltpu.sync_copy(acc.at[pl.ds(s * 640, 640)],
                        out_hbm.at[c, pl.ds(s * 640, 640)])

    return k(e4, jnp.zeros((128, 128), jnp.float32))


def _agg_kernel(hs, e4):
    @functools.partial(
        pl.kernel,
        mesh=_mesh(),
        out_type=jax.ShapeDtypeStruct((2, NP, C), jnp.float32),
        scratch_types=[
            pltpu.VMEM_SHARED((NP, C), jnp.float32),
            pltpu.VMEM((128, C), jnp.float32),
            pltpu.VMEM((128, C), jnp.float32),
            pltpu.VMEM((40, 1, 128), jnp.int32),
            pltpu.VMEM((40, 1, 128), jnp.int32),
            pltpu.SemaphoreType.DMA,
            pltpu.SemaphoreType.DMA,
            pltpu.SemaphoreType.DMA,
            pltpu.SemaphoreType.DMA,
        ],
    )
    def k(hs_hbm, ei_hbm, out_hbm, acc, rows0, rows1,
          sidx, didx, sg0, sg1, ss0, ss1):
        c = lax.axis_index("c")
        s = lax.axis_index("s")
        rows = (rows0, rows1)
        sg = (sg0, sg1)
        ss = (ss0, ss1)
        zero16 = jnp.zeros((16,), jnp.float32)

        def zb(i, _):
            rows0[i // 8, pl.ds((i % 8) * 16, 16)] = zero16
            return 0

        lax.fori_loop(0, 1024, zb, 0)
        for j in range(5):
            pltpu.sync_copy(rows0, acc.at[pl.ds(s * 640 + j * 128, 128)])
        plsc.subcore_barrier()

        base = c * (ER // 2) + s * ROWS_T

        def gath(j, b):
            pltpu.async_copy(hs_hbm.at[sidx.at[j, 0]], rows[b], sg[b])

        def scat(j, b):
            pltpu.async_copy(rows[b], acc.at[didx.at[j, 0]], ss[b], add=True)

        def wait_g(b):
            pltpu.make_async_copy(hs_hbm.at[pl.ds(0, 128)], rows[b],
                                  sg[b]).wait()

        def wait_s(b):
            pltpu.make_async_copy(hs_hbm.at[pl.ds(0, 128)], rows[b],
                                  ss[b]).wait()

        def phase(row_base, nrows):
            pltpu.sync_copy(ei_hbm.at[0, pl.ds(base + row_base, nrows)],
                            sidx.at[pl.ds(0, nrows)])
            pltpu.sync_copy(ei_hbm.at[1, pl.ds(base + row_base, nrows)],
                            didx.at[pl.ds(0, nrows)])
            gath(0, 0)
            gath(1, 1)

            def pair(j2, _):
                j = 2 * j2
                wait_g(0)
                scat(j, 0)
                wait_s(0)
                gath(j + 2, 0)
                wait_g(1)
                scat(j + 1, 1)
                wait_s(1)
                gath(j + 3, 1)
                return 0

            lax.fori_loop(0, nrows // 2 - 1, pair, 0)
            wait_g(0)
            scat(nrows - 2, 0)
            wait_g(1)
            scat(nrows - 1, 1)
            wait_s(0)
            wait_s(1)

        phase(0, 40)
        phase(40, 38)

        @pl.when(s < 2)
        def _():
            xr = c * (ER // 2) + 16 * ROWS_T + s
            pltpu.sync_copy(ei_hbm.at[0, xr], sidx.at[0])
            pltpu.sync_copy(ei_hbm.at[1, xr], didx.at[0])
            gath(0, 0)
            wait_g(0)
            scat(0, 0)
            wait_s(0)

        plsc.subcore_barrier()
        pltpu.sync_copy(acc.at[pl.ds(s * 640, 640)],
                        out_hbm.at[c, pl.ds(s * 640, 640)])

    return k(hs, e4)


def _dinv_of(d_ref):
    return lax.rsqrt(1.0 + d_ref[0, :, 0:1] + d_ref[1, :, 0:1])


def _tc_a1(x_pad, W1):
    def body(x_ref, w_ref, o_ref):
        o_ref[...] = jnp.dot(x_ref[...], w_ref[...],
                             preferred_element_type=jnp.float32)

    return pl.pallas_call(
        body,
        grid=(NP // RB,),
        in_specs=[
            pl.BlockSpec((RB, C), lambda i: (i, 0)),
            pl.BlockSpec((C, C), lambda i: (0, 0)),
        ],
        out_specs=pl.BlockSpec((RB, C), lambda i: (i, 0)),
        out_shape=jax.ShapeDtypeStruct((NP, C), jnp.float32),
    )(x_pad, W1)


def _tc_a2(h1, degp):
    def body(h_ref, d_ref, o_ref, v_ref):
        dinv = _dinv_of(d_ref)
        o_ref[...] = h_ref[...] * dinv
        v_ref[...] = jnp.broadcast_to(dinv, (RB, C))

    return pl.pallas_call(
        body,
        grid=(NP // RB,),
        in_specs=[
            pl.BlockSpec((RB, C), lambda i: (i, 0)),
            pl.BlockSpec((2, RB, C), lambda i: (0, i, 0)),
        ],
        out_specs=[pl.BlockSpec((RB, C), lambda i: (i, 0)),
                   pl.BlockSpec((RB, C), lambda i: (i, 0))],
        out_shape=[jax.ShapeDtypeStruct((NP, C), jnp.float32),
                   jax.ShapeDtypeStruct((NP, C), jnp.float32)],
    )(h1, degp)


def _tc_b(agg, h1s, dinvb, b1, W2):
    def body(a_ref, h_ref, d_ref, b_ref, w_ref, o_ref):
        dinv = d_ref[...]
        t = (a_ref[0] + a_ref[1] + h_ref[...]) * dinv + b_ref[...]
        t = jnp.maximum(t, 0.0)
        o_ref[...] = jnp.dot(t, w_ref[...],
                             preferred_element_type=jnp.float32) * dinv

    return pl.pallas_call(
        body,
        grid=(NP // RB,),
        in_specs=[
            pl.BlockSpec((2, RB, C), lambda i: (0, i, 0)),
            pl.BlockSpec((RB, C), lambda i: (i, 0)),
            pl.BlockSpec((RB, C), lambda i: (i, 0)),
            pl.BlockSpec((1, C), lambda i: (0, 0)),
            pl.BlockSpec((C, C), lambda i: (0, 0)),
        ],
        out_specs=pl.BlockSpec((RB, C), lambda i: (i, 0)),
        out_shape=jax.ShapeDtypeStruct((NP, C), jnp.float32),
    )(agg, h1s, dinvb, b1, W2)


def _tc_c(agg, h2s, dinvb, b2):
    def body(a_ref, h_ref, d_ref, b_ref, o_ref):
        o_ref[...] = (a_ref[0] + a_ref[1] + h_ref[...]) * d_ref[...] + b_ref[...]

    return pl.pallas_call(
        body,
        grid=(NP // RB,),
        in_specs=[
            pl.BlockSpec((2, RB, C), lambda i: (0, i, 0)),
            pl.BlockSpec((RB, C), lambda i: (i, 0)),
            pl.BlockSpec((RB, C), lambda i: (i, 0)),
            pl.BlockSpec((1, C), lambda i: (0, 0)),
        ],
        out_specs=pl.BlockSpec((RB, C), lambda i: (i, 0)),
        out_shape=jax.ShapeDtypeStruct((NP, C), jnp.float32),
    )(agg, h2s, dinvb, b2)


def kernel(x, edge_index, W1, b1, W2, b2):
    e4 = edge_index.astype(jnp.int32).reshape(2, ER, 1, 128)
    x_pad = jnp.pad(x, ((0, NP - N), (0, 0)))

    h1 = _tc_a1(x_pad, W1)
    degp = _deg_kernel(e4)
    h1s, dinvb = _tc_a2(h1, degp)
    agg1 = _agg_kernel(h1s, e4)
    h2s = _tc_b(agg1, h1s, dinvb, b1.reshape(1, C), W2)
    agg2 = _agg_kernel(h2s, e4)
    outp = _tc_c(agg2, h2s, dinvb, b2.reshape(1, C))
    return outp[:N]


# speedup vs baseline: 1.0273x; 1.0273x over previous
import functools

import jax
import jax.numpy as jnp
from jax import lax
from jax.experimental import pallas as pl
from jax.experimental.pallas import tpu as pltpu
from jax.experimental.pallas import tpu_sc as plsc

N = 10000
NP = 10000
E = 320000
ER = E // 128
C = 128
RB = 2000


def _mesh():
    return plsc.VectorSubcoreMesh(core_axis_name="c", subcore_axis_name="s")


ROWS_T = 78
CH = 3
NCH = ROWS_T // CH
NPAIR = NCH // 2 - 1


def _deg_kernel(e4):
    @functools.partial(
        pl.kernel,
        mesh=_mesh(),
        out_type=jax.ShapeDtypeStruct((2, NP, C), jnp.float32),
        scratch_types=[
            pltpu.VMEM_SHARED((NP, C), jnp.float32),
            pltpu.VMEM((128, C), jnp.float32),
            pltpu.VMEM((80, 1, 128), jnp.int32),
            pltpu.SemaphoreType.DMA,
            pltpu.SemaphoreType.DMA,
        ],
    )
    def k(ei_hbm, zdum_hbm, out_hbm, acc, buf, didx, sem0, sem1):
        c = lax.axis_index("c")
        s = lax.axis_index("s")
        sems = (sem0, sem1)
        zeroi = jnp.zeros((16,), jnp.float32)
        onesi = jnp.ones((16,), jnp.float32)

        def zb(i, _):
            buf[i // 8, pl.ds((i % 8) * 16, 16)] = zeroi
            return 0

        lax.fori_loop(0, 1024, zb, 0)
        for j in range(4):
            pltpu.sync_copy(buf, acc.at[pl.ds(s * 624 + j * 128, 128)])
        pltpu.sync_copy(buf.at[pl.ds(0, 112)],
                        acc.at[pl.ds(s * 624 + 512, 112)])

        @pl.when(s == 0)
        def _():
            pltpu.sync_copy(buf.at[pl.ds(0, 16)], acc.at[pl.ds(9984, 16)])

        def ob(i, _):
            buf[i // 8, pl.ds((i % 8) * 16, 16)] = onesi
            return 0

        lax.fori_loop(0, 1024, ob, 0)
        base = c * (ER // 2) + s * ROWS_T
        pltpu.sync_copy(ei_hbm.at[1, pl.ds(base, ROWS_T)],
                        didx.at[pl.ds(0, ROWS_T)])

        @pl.when(s < 2)
        def _():
            pltpu.sync_copy(ei_hbm.at[1, c * (ER // 2) + 16 * ROWS_T + s],
                            didx.at[ROWS_T])

        plsc.subcore_barrier()

        def scat(j, b):
            for kk in range(CH):
                pltpu.async_copy(buf, acc.at[didx.at[j * CH + kk, 0]],
                                 sems[b], add=True)

        def wait_s(b):
            for kk in range(CH):
                pltpu.make_async_copy(zdum_hbm.at[pl.ds(0, 128)], buf,
                                      sems[b]).wait()

        scat(0, 0)
        scat(1, 1)

        def pair(j2, _):
            j = 2 * j2
            wait_s(0)
            scat(j + 2, 0)
            wait_s(1)
            scat(j + 3, 1)
            return 0

        lax.fori_loop(0, NPAIR, pair, 0)
        wait_s(0)
        wait_s(1)

        @pl.when(s < 2)
        def _():
            pltpu.async_copy(buf, acc.at[didx.at[ROWS_T, 0]], sem0, add=True)
            pltpu.make_async_copy(zdum_hbm.at[pl.ds(0, 128)], buf,
                                  sem0).wait()

        plsc.subcore_barrier()
        pltpu.sync_copy(acc.at[pl.ds(s * 624, 624)],
                        out_hbm.at[c, pl.ds(s * 624, 624)])

        @pl.when(s == 0)
        def _():
            pltpu.sync_copy(acc.at[pl.ds(9984, 16)],
                            out_hbm.at[c, pl.ds(9984, 16)])

    return k(e4, jnp.zeros((128, 128), jnp.float32))


def _agg_kernel(hs, e4):
    @functools.partial(
        pl.kernel,
        mesh=_mesh(),
        out_type=jax.ShapeDtypeStruct((2, NP, C), jnp.float32),
        scratch_types=[
            pltpu.VMEM_SHARED((NP, C), jnp.float32),
            pltpu.VMEM((128, C), jnp.float32),
            pltpu.VMEM((128, C), jnp.float32),
            pltpu.VMEM((40, 1, 128), jnp.int32),
            pltpu.VMEM((40, 1, 128), jnp.int32),
            pltpu.SemaphoreType.DMA,
            pltpu.SemaphoreType.DMA,
            pltpu.SemaphoreType.DMA,
            pltpu.SemaphoreType.DMA,
        ],
    )
    def k(hs_hbm, ei_hbm, out_hbm, acc, rows0, rows1,
          sidx, didx, sg0, sg1, ss0, ss1):
        c = lax.axis_index("c")
        s = lax.axis_index("s")
        rows = (rows0, rows1)
        sg = (sg0, sg1)
        ss = (ss0, ss1)
        zero16 = jnp.zeros((16,), jnp.float32)

        def zb(i, _):
            rows0[i // 8, pl.ds((i % 8) * 16, 16)] = zero16
            return 0

        lax.fori_loop(0, 1024, zb, 0)
        for j in range(4):
            pltpu.sync_copy(rows0, acc.at[pl.ds(s * 624 + j * 128, 128)])
        pltpu.sync_copy(rows0.at[pl.ds(0, 112)],
                        acc.at[pl.ds(s * 624 + 512, 112)])

        @pl.when(s == 0)
        def _():
            pltpu.sync_copy(rows0.at[pl.ds(0, 16)], acc.at[pl.ds(9984, 16)])

        plsc.subcore_barrier()

        base = c * (ER // 2) + s * ROWS_T

        def gath(j, b):
            pltpu.async_copy(hs_hbm.at[sidx.at[j, 0]], rows[b], sg[b])

        def scat(j, b):
            pltpu.async_copy(rows[b], acc.at[didx.at[j, 0]], ss[b], add=True)

        def wait_g(b):
            pltpu.make_async_copy(hs_hbm.at[pl.ds(0, 128)], rows[b],
                                  sg[b]).wait()

        def wait_s(b):
            pltpu.make_async_copy(hs_hbm.at[pl.ds(0, 128)], rows[b],
                                  ss[b]).wait()

        def phase(row_base, nrows):
            pltpu.sync_copy(ei_hbm.at[0, pl.ds(base + row_base, nrows)],
                            sidx.at[pl.ds(0, nrows)])
            pltpu.sync_copy(ei_hbm.at[1, pl.ds(base + row_base, nrows)],
                            didx.at[pl.ds(0, nrows)])
            gath(0, 0)
            gath(1, 1)

            def pair(j2, _):
                j = 2 * j2
                wait_g(0)
                scat(j, 0)
                wait_s(0)
                gath(j + 2, 0)
                wait_g(1)
                scat(j + 1, 1)
                wait_s(1)
                gath(j + 3, 1)
                return 0

            lax.fori_loop(0, nrows // 2 - 1, pair, 0)
            wait_g(0)
            scat(nrows - 2, 0)
            wait_g(1)
            scat(nrows - 1, 1)
            wait_s(0)
            wait_s(1)

        phase(0, 40)
        phase(40, 38)

        @pl.when(s < 2)
        def _():
            xr = c * (ER // 2) + 16 * ROWS_T + s
            pltpu.sync_copy(ei_hbm.at[0, xr], sidx.at[0])
            pltpu.sync_copy(ei_hbm.at[1, xr], didx.at[0])
            gath(0, 0)
            wait_g(0)
            scat(0, 0)
            wait_s(0)

        plsc.subcore_barrier()
        pltpu.sync_copy(acc.at[pl.ds(s * 624, 624)],
                        out_hbm.at[c, pl.ds(s * 624, 624)])

        @pl.when(s == 0)
        def _():
            pltpu.sync_copy(acc.at[pl.ds(9984, 16)],
                            out_hbm.at[c, pl.ds(9984, 16)])

    return k(hs, e4)


def _dinv_of(d_ref):
    return lax.rsqrt(1.0 + d_ref[0, :, 0:1] + d_ref[1, :, 0:1])


def _tc_a1(x_pad, W1):
    def body(x_ref, w_ref, o_ref):
        o_ref[...] = jnp.dot(x_ref[...], w_ref[...],
                             preferred_element_type=jnp.float32)

    return pl.pallas_call(
        body,
        grid=(NP // RB,),
        in_specs=[
            pl.BlockSpec((RB, C), lambda i: (i, 0)),
            pl.BlockSpec((C, C), lambda i: (0, 0)),
        ],
        out_specs=pl.BlockSpec((RB, C), lambda i: (i, 0)),
        out_shape=jax.ShapeDtypeStruct((NP, C), jnp.float32),
    )(x_pad, W1)


def _tc_a2(h1, degp):
    def body(h_ref, d_ref, o_ref, v_ref):
        dinv = _dinv_of(d_ref)
        o_ref[...] = h_ref[...] * dinv
        v_ref[...] = jnp.broadcast_to(dinv, (RB, C))

    return pl.pallas_call(
        body,
        grid=(NP // RB,),
        in_specs=[
            pl.BlockSpec((RB, C), lambda i: (i, 0)),
            pl.BlockSpec((2, RB, C), lambda i: (0, i, 0)),
        ],
        out_specs=[pl.BlockSpec((RB, C), lambda i: (i, 0)),
                   pl.BlockSpec((RB, C), lambda i: (i, 0))],
        out_shape=[jax.ShapeDtypeStruct((NP, C), jnp.float32),
                   jax.ShapeDtypeStruct((NP, C), jnp.float32)],
    )(h1, degp)


def _tc_b(agg, h1s, dinvb, b1, W2):
    def body(a_ref, h_ref, d_ref, b_ref, w_ref, o_ref):
        dinv = d_ref[...]
        t = (a_ref[0] + a_ref[1] + h_ref[...]) * dinv + b_ref[...]
        t = jnp.maximum(t, 0.0)
        o_ref[...] = jnp.dot(t, w_ref[...],
                             preferred_element_type=jnp.float32) * dinv

    return pl.pallas_call(
        body,
        grid=(NP // RB,),
        in_specs=[
            pl.BlockSpec((2, RB, C), lambda i: (0, i, 0)),
            pl.BlockSpec((RB, C), lambda i: (i, 0)),
            pl.BlockSpec((RB, C), lambda i: (i, 0)),
            pl.BlockSpec((1, C), lambda i: (0, 0)),
            pl.BlockSpec((C, C), lambda i: (0, 0)),
        ],
        out_specs=pl.BlockSpec((RB, C), lambda i: (i, 0)),
        out_shape=jax.ShapeDtypeStruct((NP, C), jnp.float32),
    )(agg, h1s, dinvb, b1, W2)


def _tc_c(agg, h2s, dinvb, b2):
    def body(a_ref, h_ref, d_ref, b_ref, o_ref):
        o_ref[...] = (a_ref[0] + a_ref[1] + h_ref[...]) * d_ref[...] + b_ref[...]

    return pl.pallas_call(
        body,
        grid=(NP // RB,),
        in_specs=[
            pl.BlockSpec((2, RB, C), lambda i: (0, i, 0)),
            pl.BlockSpec((RB, C), lambda i: (i, 0)),
            pl.BlockSpec((RB, C), lambda i: (i, 0)),
            pl.BlockSpec((1, C), lambda i: (0, 0)),
        ],
        out_specs=pl.BlockSpec((RB, C), lambda i: (i, 0)),
        out_shape=jax.ShapeDtypeStruct((NP, C), jnp.float32),
    )(agg, h2s, dinvb, b2)


def kernel(x, edge_index, W1, b1, W2, b2):
    e4 = edge_index.astype(jnp.int32).reshape(2, ER, 1, 128)

    h1 = _tc_a1(x, W1)
    degp = _deg_kernel(e4)
    h1s, dinvb = _tc_a2(h1, degp)
    agg1 = _agg_kernel(h1s, e4)
    h2s = _tc_b(agg1, h1s, dinvb, b1.reshape(1, C), W2)
    agg2 = _agg_kernel(h2s, e4)
    outp = _tc_c(agg2, h2s, dinvb, b2.reshape(1, C))
    return outp


# speedup vs baseline: 1.0286x; 1.0013x over previous
import functools

import jax
import jax.numpy as jnp
from jax import lax
from jax.experimental import pallas as pl
from jax.experimental.pallas import tpu as pltpu
from jax.experimental.pallas import tpu_sc as plsc

N = 10000
NP = 10000
E = 320000
ER = E // 128
C = 128
RB = 2000


def _mesh():
    return plsc.VectorSubcoreMesh(core_axis_name="c", subcore_axis_name="s")


ROWS_T = 78
CH = 3
NCH = ROWS_T // CH
NPAIR = NCH // 2 - 1


def _deg_kernel(e4):
    @functools.partial(
        pl.kernel,
        mesh=_mesh(),
        out_type=jax.ShapeDtypeStruct((2, NP, C), jnp.float32),
        scratch_types=[
            pltpu.VMEM_SHARED((NP, C), jnp.float32),
            pltpu.VMEM((128, C), jnp.float32),
            pltpu.VMEM((80, 1, 128), jnp.int32),
            pltpu.SemaphoreType.DMA,
            pltpu.SemaphoreType.DMA,
        ],
    )
    def k(ei_hbm, zdum_hbm, out_hbm, acc, buf, didx, sem0, sem1):
        c = lax.axis_index("c")
        s = lax.axis_index("s")
        sems = (sem0, sem1)
        zeroi = jnp.zeros((16,), jnp.float32)
        onesi = jnp.ones((16,), jnp.float32)

        def zb(i, _):
            buf[i // 8, pl.ds((i % 8) * 16, 16)] = zeroi
            return 0

        lax.fori_loop(0, 1024, zb, 0)
        for j in range(4):
            pltpu.sync_copy(buf, acc.at[pl.ds(s * 624 + j * 128, 128)])
        pltpu.sync_copy(buf.at[pl.ds(0, 112)],
                        acc.at[pl.ds(s * 624 + 512, 112)])

        @pl.when(s == 0)
        def _():
            pltpu.sync_copy(buf.at[pl.ds(0, 16)], acc.at[pl.ds(9984, 16)])

        def ob(i, _):
            buf[i // 8, pl.ds((i % 8) * 16, 16)] = onesi
            return 0

        lax.fori_loop(0, 1024, ob, 0)
        base = c * (ER // 2) + s * ROWS_T
        pltpu.sync_copy(ei_hbm.at[1, pl.ds(base, ROWS_T)],
                        didx.at[pl.ds(0, ROWS_T)])

        @pl.when(s < 2)
        def _():
            pltpu.sync_copy(ei_hbm.at[1, c * (ER // 2) + 16 * ROWS_T + s],
                            didx.at[ROWS_T])

        plsc.subcore_barrier()

        def scat(j, b):
            for kk in range(CH):
                pltpu.async_copy(buf, acc.at[didx.at[j * CH + kk, 0]],
                                 sems[b], add=True)

        def wait_s(b):
            for kk in range(CH):
                pltpu.make_async_copy(zdum_hbm.at[pl.ds(0, 128)], buf,
                                      sems[b]).wait()

        scat(0, 0)
        scat(1, 1)

        def pair(j2, _):
            j = 2 * j2
            wait_s(0)
            scat(j + 2, 0)
            wait_s(1)
            scat(j + 3, 1)
            return 0

        lax.fori_loop(0, NPAIR, pair, 0)
        wait_s(0)
        wait_s(1)

        @pl.when(s < 2)
        def _():
            pltpu.async_copy(buf, acc.at[didx.at[ROWS_T, 0]], sem0, add=True)
            pltpu.make_async_copy(zdum_hbm.at[pl.ds(0, 128)], buf,
                                  sem0).wait()

        plsc.subcore_barrier()
        pltpu.sync_copy(acc.at[pl.ds(s * 624, 624)],
                        out_hbm.at[c, pl.ds(s * 624, 624)])

        @pl.when(s == 0)
        def _():
            pltpu.sync_copy(acc.at[pl.ds(9984, 16)],
                            out_hbm.at[c, pl.ds(9984, 16)])

    return k(e4, jnp.zeros((128, 128), jnp.float32))


def _agg_kernel(hs, e4):
    @functools.partial(
        pl.kernel,
        mesh=_mesh(),
        out_type=jax.ShapeDtypeStruct((2, NP, C), jnp.float32),
        scratch_types=[
            pltpu.VMEM_SHARED((NP, C), jnp.float32),
            pltpu.VMEM((128, C), jnp.float32),
            pltpu.VMEM((128, C), jnp.float32),
            pltpu.VMEM((40, 1, 128), jnp.int32),
            pltpu.VMEM((40, 1, 128), jnp.int32),
            pltpu.SemaphoreType.DMA,
            pltpu.SemaphoreType.DMA,
            pltpu.SemaphoreType.DMA,
            pltpu.SemaphoreType.DMA,
        ],
    )
    def k(hs_hbm, ei_hbm, out_hbm, acc, rows0, rows1,
          sidx, didx, sg0, sg1, ss0, ss1):
        c = lax.axis_index("c")
        s = lax.axis_index("s")
        rows = (rows0, rows1)
        sg = (sg0, sg1)
        ss = (ss0, ss1)
        zero16 = jnp.zeros((16,), jnp.float32)

        def zb(i, _):
            rows0[i // 8, pl.ds((i % 8) * 16, 16)] = zero16
            return 0

        lax.fori_loop(0, 1024, zb, 0)
        for j in range(4):
            pltpu.sync_copy(rows0, acc.at[pl.ds(s * 624 + j * 128, 128)])
        pltpu.sync_copy(rows0.at[pl.ds(0, 112)],
                        acc.at[pl.ds(s * 624 + 512, 112)])

        @pl.when(s == 0)
        def _():
            pltpu.sync_copy(rows0.at[pl.ds(0, 16)], acc.at[pl.ds(9984, 16)])

        plsc.subcore_barrier()

        base = c * (ER // 2) + s * ROWS_T

        def gath(j, b):
            pltpu.async_copy(hs_hbm.at[sidx.at[j, 0]], rows[b], sg[b])

        def scat(j, b):
            pltpu.async_copy(rows[b], acc.at[didx.at[j, 0]], ss[b], add=True)

        def wait_g(b):
            pltpu.make_async_copy(hs_hbm.at[pl.ds(0, 128)], rows[b],
                                  sg[b]).wait()

        def wait_s(b):
            pltpu.make_async_copy(hs_hbm.at[pl.ds(0, 128)], rows[b],
                                  ss[b]).wait()

        def phase(row_base, nrows):
            pltpu.sync_copy(ei_hbm.at[0, pl.ds(base + row_base, nrows)],
                            sidx.at[pl.ds(0, nrows)])
            pltpu.sync_copy(ei_hbm.at[1, pl.ds(base + row_base, nrows)],
                            didx.at[pl.ds(0, nrows)])
            gath(0, 0)
            gath(1, 1)

            def pair(j2, _):
                j = 2 * j2
                wait_g(0)
                scat(j, 0)
                wait_s(0)
                gath(j + 2, 0)
                wait_g(1)
                scat(j + 1, 1)
                wait_s(1)
                gath(j + 3, 1)
                return 0

            lax.fori_loop(0, nrows // 2 - 1, pair, 0)
            wait_g(0)
            scat(nrows - 2, 0)
            wait_g(1)
            scat(nrows - 1, 1)
            wait_s(0)
            wait_s(1)

        phase(0, 40)
        phase(40, 38)

        @pl.when(s < 2)
        def _():
            xr = c * (ER // 2) + 16 * ROWS_T + s
            pltpu.sync_copy(ei_hbm.at[0, xr], sidx.at[0])
            pltpu.sync_copy(ei_hbm.at[1, xr], didx.at[0])
            gath(0, 0)
            wait_g(0)
            scat(0, 0)
            wait_s(0)

        plsc.subcore_barrier()
        pltpu.sync_copy(acc.at[pl.ds(s * 624, 624)],
                        out_hbm.at[c, pl.ds(s * 624, 624)])

        @pl.when(s == 0)
        def _():
            pltpu.sync_copy(acc.at[pl.ds(9984, 16)],
                            out_hbm.at[c, pl.ds(9984, 16)])

    return k(hs, e4)


def _dinv_of(d_ref):
    return lax.rsqrt(1.0 + d_ref[0, :, 0:1] + d_ref[1, :, 0:1])


def _tc_a(x, W1, degp):
    def body(x_ref, w_ref, d_ref, o_ref, v_ref):
        dinv = _dinv_of(d_ref)
        h = jnp.dot(x_ref[...], w_ref[...], preferred_element_type=jnp.float32)
        o_ref[...] = h * dinv
        v_ref[...] = jnp.broadcast_to(dinv, (RB, C))

    return pl.pallas_call(
        body,
        grid=(NP // RB,),
        in_specs=[
            pl.BlockSpec((RB, C), lambda i: (i, 0)),
            pl.BlockSpec((C, C), lambda i: (0, 0)),
            pl.BlockSpec((2, RB, C), lambda i: (0, i, 0)),
        ],
        out_specs=[pl.BlockSpec((RB, C), lambda i: (i, 0)),
                   pl.BlockSpec((RB, C), lambda i: (i, 0))],
        out_shape=[jax.ShapeDtypeStruct((NP, C), jnp.float32),
                   jax.ShapeDtypeStruct((NP, C), jnp.float32)],
    )(x, W1, degp)


def _tc_b(agg, h1s, dinvb, b1, W2):
    def body(a_ref, h_ref, d_ref, b_ref, w_ref, o_ref):
        dinv = d_ref[...]
        t = (a_ref[0] + a_ref[1] + h_ref[...]) * dinv + b_ref[...]
        t = jnp.maximum(t, 0.0)
        o_ref[...] = jnp.dot(t, w_ref[...],
                             preferred_element_type=jnp.float32) * dinv

    return pl.pallas_call(
        body,
        grid=(NP // RB,),
        in_specs=[
            pl.BlockSpec((2, RB, C), lambda i: (0, i, 0)),
            pl.BlockSpec((RB, C), lambda i: (i, 0)),
            pl.BlockSpec((RB, C), lambda i: (i, 0)),
            pl.BlockSpec((1, C), lambda i: (0, 0)),
            pl.BlockSpec((C, C), lambda i: (0, 0)),
        ],
        out_specs=pl.BlockSpec((RB, C), lambda i: (i, 0)),
        out_shape=jax.ShapeDtypeStruct((NP, C), jnp.float32),
    )(agg, h1s, dinvb, b1, W2)


def _tc_c(agg, h2s, dinvb, b2):
    def body(a_ref, h_ref, d_ref, b_ref, o_ref):
        o_ref[...] = (a_ref[0] + a_ref[1] + h_ref[...]) * d_ref[...] + b_ref[...]

    return pl.pallas_call(
        body,
        grid=(NP // RB,),
        in_specs=[
            pl.BlockSpec((2, RB, C), lambda i: (0, i, 0)),
            pl.BlockSpec((RB, C), lambda i: (i, 0)),
            pl.BlockSpec((RB, C), lambda i: (i, 0)),
            pl.BlockSpec((1, C), lambda i: (0, 0)),
        ],
        out_specs=pl.BlockSpec((RB, C), lambda i: (i, 0)),
        out_shape=jax.ShapeDtypeStruct((NP, C), jnp.float32),
    )(agg, h2s, dinvb, b2)


def kernel(x, edge_index, W1, b1, W2, b2):
    e4 = edge_index.astype(jnp.int32).reshape(2, ER, 1, 128)

    degp = _deg_kernel(e4)
    h1s, dinvb = _tc_a(x, W1, degp)
    agg1 = _agg_kernel(h1s, e4)
    h2s = _tc_b(agg1, h1s, dinvb, b1.reshape(1, C), W2)
    agg2 = _agg_kernel(h2s, e4)
    outp = _tc_c(agg2, h2s, dinvb, b2.reshape(1, C))
    return outp
